# Initial kernel scaffold; baseline (speedup 1.0000x reference)
import jax
import jax.numpy as jnp
from jax.experimental import pallas as pl


def kernel(x, edge_index, batch, W1l, b1l, W1r, W2l, b2l, W2r, Wa1, ba1, Wa2, ba2, Wc1, bc1, Wc2, bc2):
    raise NotImplementedError("write your pallas kernel here")


# speedup vs baseline: 8.7576x; 8.7576x over previous
import functools

import jax
import jax.numpy as jnp
from jax import lax
from jax.experimental import pallas as pl
from jax.experimental.pallas import tpu as pltpu
from jax.experimental.pallas import tpu_sc as plsc

_N_CORES = 2
_N_SUB = 16
_G = 64

_PREC = jax.lax.Precision.HIGHEST


def _dot(a, b):
    return jax.lax.dot_general(a, b, (((1,), (0,)), ((), ())),
                               precision=_PREC,
                               preferred_element_type=jnp.float32)


@functools.lru_cache(maxsize=None)
def _make_sc_pass(N, E, W, C):
    NW = _N_CORES * _N_SUB
    EPW = E // NW
    NCH = EPW // C
    RPT = (N // _N_SUB) // 8 * 8
    TAIL = N - _N_SUB * RPT
    assert EPW * NW == E and NCH * C == EPW and 0 <= TAIL <= RPT
    assert TAIL % 8 == 0 and (_N_SUB * RPT) % 8 == 0

    mesh = plsc.VectorSubcoreMesh(core_axis_name="c", subcore_axis_name="s")

    @functools.partial(
        pl.kernel,
        out_type=jax.ShapeDtypeStruct((_N_CORES, N, W), jnp.float32),
        mesh=mesh,
        scratch_types=[
            pltpu.VMEM((NCH, C), jnp.int32),
            pltpu.VMEM((NCH, C), jnp.int32),
            pltpu.VMEM((C, W), jnp.float32),
            pltpu.VMEM((RPT, W), jnp.float32),
            pltpu.VMEM_SHARED((N, W), jnp.float32),
            pltpu.SemaphoreType.DMA,
        ],
        compiler_params=pltpu.CompilerParams(use_tc_tiling_on_sc=False),
    )
    def sc_pass(u_hbm, src_hbm, dst_hbm, out_hbm, sidx, didx, rows, zbuf,
                acc, sem):
        c = lax.axis_index("c")
        s = lax.axis_index("s")
        w = c * _N_SUB + s

        zero = jnp.zeros((16,), jnp.float32)

        def zrow(i, carry):
            for k in range(W // 16):
                zbuf[i, pl.ds(k * 16, 16)] = zero
            return carry

        lax.fori_loop(0, RPT, zrow, 0)
        pltpu.sync_copy(zbuf, acc.at[pl.ds(s * RPT, RPT)])
        if TAIL:
            @pl.when(s == _N_SUB - 1)
            def _():
                pltpu.sync_copy(zbuf.at[pl.ds(0, TAIL)],
                                acc.at[pl.ds(_N_SUB * RPT, TAIL)])
        pltpu.sync_copy(src_hbm.at[w], sidx)
        pltpu.sync_copy(dst_hbm.at[w], didx)
        plsc.subcore_barrier()

        def body(i, carry):
            pltpu.async_copy(u_hbm.at[sidx.at[i]], rows, sem).wait()
            pltpu.sync_copy(rows, acc.at[didx.at[i]], add=True)
            return carry

        lax.fori_loop(0, NCH, body, 0)
        plsc.subcore_barrier()
        pltpu.sync_copy(acc.at[pl.ds(s * RPT, RPT)],
                        out_hbm.at[c, pl.ds(s * RPT, RPT)])
        if TAIL:
            @pl.when(s == _N_SUB - 1)
            def _():
                pltpu.sync_copy(acc.at[pl.ds(_N_SUB * RPT, TAIL)],
                                out_hbm.at[c, pl.ds(_N_SUB * RPT, TAIL)])

    return sc_pass


def _tc_pre_body(x_ref, wl_ref, wr_ref, u_ref, r_ref):
    x = x_ref[...]
    u = _dot(x, wl_ref[...])
    tail = (lax.broadcasted_iota(jnp.int32, (x.shape[0], 16), 1) == 0)
    u_ref[...] = jnp.concatenate([u, tail.astype(jnp.float32)], axis=1)
    r_ref[...] = _dot(x, wr_ref[...])


def _tc_mid_body(sa_ref, sb_ref, r1_ref, b_ref, wl_ref, wr_ref,
                 u2_ref, r2_ref):
    sa = sa_ref[...]
    sb = sb_ref[...]
    ssum = sa[:, :64] + sb[:, :64]
    cnt = sa[:, 64:65] + sb[:, 64:65]
    inv = 1.0 / jnp.maximum(cnt, 1.0)
    h1 = jnp.maximum(ssum * inv + r1_ref[...] + b_ref[...], 0.0)
    u2_ref[...] = _dot(h1, wl_ref[...])
    r2 = _dot(h1, wr_ref[...])
    r2_ref[...] = jnp.concatenate(
        [r2, jnp.broadcast_to(inv, r2.shape)], axis=1)


def _tc_post_body(sa_ref, sb_ref, r2_ref, b_ref, bt_ref,
                  wa1_ref, ba1_ref, wa2_ref, ba2_ref,
                  wc1_ref, bc1_ref, wc2_ref, bc2_ref,
                  mu_ref, val_ref):
    r2a = r2_ref[...]
    inv = r2a[:, 64:65]
    h2 = jnp.maximum((sa_ref[...] + sb_ref[...]) * inv + r2a[:, :64]
                     + b_ref[...], 0.0)
    n = h2.shape[0]
    bt = jnp.broadcast_to(bt_ref[...], (_G, n))
    ohT = (bt == lax.broadcasted_iota(jnp.int32, (_G, n), 0)) \
        .astype(jnp.float32)
    sums = _dot(ohT, h2)
    cnts = jnp.sum(ohT, axis=1, keepdims=True)
    pooled = sums / jnp.maximum(cnts, 1.0)
    a = jnp.maximum(_dot(pooled, wa1_ref[...]) + ba1_ref[...], 0.0)
    mu_ref[...] = _dot(a, wa2_ref[...]) + ba2_ref[...]
    cv = jnp.maximum(_dot(pooled, wc1_ref[...]) + bc1_ref[...], 0.0)
    val_ref[...] = _dot(cv, wc2_ref[...]) + bc2_ref[...]


def kernel(x, edge_index, batch, W1l, b1l, W1r, W2l, b2l, W2r,
           Wa1, ba1, Wa2, ba2, Wc1, bc1, Wc2, bc2):
    N, F = x.shape
    E = edge_index.shape[1]
    H = W1l.shape[0]
    A = Wa2.shape[0]
    W1 = H + 16
    C = 80

    f32 = jnp.float32
    nw = _N_CORES * _N_SUB
    src2d = edge_index[0].reshape(nw, E // (nw * C), C)
    dst2d = edge_index[1].reshape(nw, E // (nw * C), C)

    u1, r1 = pl.pallas_call(
        _tc_pre_body,
        out_shape=[jax.ShapeDtypeStruct((N, W1), f32),
                   jax.ShapeDtypeStruct((N, H), f32)],
    )(x, W1l.T, W1r.T)

    s1 = _make_sc_pass(N, E, W1, C)(u1, src2d, dst2d)

    u2, r2a = pl.pallas_call(
        _tc_mid_body,
        out_shape=[jax.ShapeDtypeStruct((N, H), f32),
                   jax.ShapeDtypeStruct((N, 2 * H), f32)],
    )(s1[0], s1[1], r1, b1l.reshape(1, H), W2l.T, W2r.T)

    s2 = _make_sc_pass(N, E, H, C)(u2, src2d, dst2d)

    mu, value = pl.pallas_call(
        _tc_post_body,
        out_shape=[jax.ShapeDtypeStruct((_G, A), f32),
                   jax.ShapeDtypeStruct((_G, 1), f32)],
    )(s2[0], s2[1], r2a, b2l.reshape(1, H), batch.reshape(1, N),
      Wa1.T, ba1.reshape(1, H), Wa2.T, ba2.reshape(1, A),
      Wc1.T, bc1.reshape(1, H), Wc2.T, bc2.reshape(1, 1))

    return (mu, value)


# speedup vs baseline: 16.4387x; 1.8771x over previous
import functools

import jax
import jax.numpy as jnp
from jax import lax
from jax.experimental import pallas as pl
from jax.experimental.pallas import tpu as pltpu
from jax.experimental.pallas import tpu_sc as plsc

_N_CORES = 2
_N_SUB = 16
_G = 64

_PREC = jax.lax.Precision.HIGHEST


def _dot(a, b):
    return jax.lax.dot_general(a, b, (((1,), (0,)), ((), ())),
                               precision=_PREC,
                               preferred_element_type=jnp.float32)


@functools.lru_cache(maxsize=None)
def _make_sc_pass(N, E, W, C):
    NW = _N_CORES * _N_SUB
    EPW = E // NW
    NCH = EPW // C
    RPT = (N // _N_SUB) // 8 * 8
    TAIL = N - _N_SUB * RPT
    assert EPW * NW == E and NCH * C == EPW and 0 <= TAIL <= RPT
    assert TAIL % 8 == 0 and (_N_SUB * RPT) % 8 == 0

    NBUF = 5
    ZR = RPT // 3
    assert NCH % NBUF == 0 and NCH // NBUF >= 2
    assert RPT % ZR == 0 and ZR % 8 == 0 and TAIL <= ZR
    mesh = plsc.VectorSubcoreMesh(core_axis_name="c", subcore_axis_name="s")

    @functools.partial(
        pl.kernel,
        out_type=jax.ShapeDtypeStruct((_N_CORES, N, W), jnp.float32),
        mesh=mesh,
        scratch_types=[
            pltpu.VMEM((NCH, C), jnp.int32),
            pltpu.VMEM((NCH, C), jnp.int32),
            pltpu.VMEM((NBUF, C, W), jnp.float32),
            pltpu.VMEM((ZR, W), jnp.float32),
            pltpu.VMEM_SHARED((N, W), jnp.float32),
        ] + [pltpu.SemaphoreType.DMA] * NBUF,
        compiler_params=pltpu.CompilerParams(use_tc_tiling_on_sc=False),
    )
    def sc_pass(u_hbm, src_hbm, dst_hbm, out_hbm, sidx, didx, rows, zbuf,
                acc, *sems):
        c = lax.axis_index("c")
        s = lax.axis_index("s")
        w = c * _N_SUB + s

        pltpu.async_copy(src_hbm.at[w], sidx, sems[0])
        pltpu.async_copy(dst_hbm.at[w], didx, sems[1])

        zero = jnp.zeros((16,), jnp.float32)

        def zrow(i, carry):
            for k in range(W // 16):
                zbuf[i, pl.ds(k * 16, 16)] = zero
            return carry

        lax.fori_loop(0, ZR, zrow, 0)
        for k in range(RPT // ZR):
            pltpu.sync_copy(zbuf, acc.at[pl.ds(s * RPT + k * ZR, ZR)])
        if TAIL:
            @pl.when(s == _N_SUB - 1)
            def _():
                pltpu.sync_copy(zbuf.at[pl.ds(0, TAIL)],
                                acc.at[pl.ds(_N_SUB * RPT, TAIL)])
        pltpu.make_async_copy(src_hbm.at[w], sidx, sems[0]).wait()
        pltpu.make_async_copy(dst_hbm.at[w], didx, sems[1]).wait()
        plsc.subcore_barrier()

        def gather_start(i, j):
            pltpu.async_copy(u_hbm.at[sidx.at[i]], rows.at[j], sems[j])

        def gather_wait(j):
            pltpu.make_async_copy(u_hbm.at[pl.ds(0, C)], rows.at[j],
                                  sems[j]).wait()

        for j in range(NBUF):
            gather_start(j, j)

        def body(g, carry):
            for j in range(NBUF):
                i = g * NBUF + j
                gather_wait(j)
                pltpu.sync_copy(rows.at[j], acc.at[didx.at[i]], add=True)
                gather_start(i + NBUF, j)
            return carry

        lax.fori_loop(0, NCH // NBUF - 1, body, 0)
        for j in range(NBUF):
            i = NCH - NBUF + j
            gather_wait(j)
            pltpu.sync_copy(rows.at[j], acc.at[didx.at[i]], add=True)
        plsc.subcore_barrier()
        pltpu.sync_copy(acc.at[pl.ds(s * RPT, RPT)],
                        out_hbm.at[c, pl.ds(s * RPT, RPT)])
        if TAIL:
            @pl.when(s == _N_SUB - 1)
            def _():
                pltpu.sync_copy(acc.at[pl.ds(_N_SUB * RPT, TAIL)],
                                out_hbm.at[c, pl.ds(_N_SUB * RPT, TAIL)])

    return sc_pass


def _tc_pre_body(x_ref, wl_ref, wr_ref, u_ref, r_ref):
    x = x_ref[...]
    u = _dot(x, wl_ref[...])
    tail = (lax.broadcasted_iota(jnp.int32, (x.shape[0], 16), 1) == 0)
    u_ref[...] = jnp.concatenate([u, tail.astype(jnp.float32)], axis=1)
    r_ref[...] = _dot(x, wr_ref[...])


def _tc_mid_body(sa_ref, sb_ref, r1_ref, b_ref, wl_ref, wr_ref,
                 u2_ref, r2_ref):
    sa = sa_ref[...]
    sb = sb_ref[...]
    ssum = sa[:, :64] + sb[:, :64]
    cnt = sa[:, 64:65] + sb[:, 64:65]
    inv = 1.0 / jnp.maximum(cnt, 1.0)
    h1 = jnp.maximum(ssum * inv + r1_ref[...] + b_ref[...], 0.0)
    u2_ref[...] = _dot(h1, wl_ref[...])
    r2 = _dot(h1, wr_ref[...])
    r2_ref[...] = jnp.concatenate(
        [r2, jnp.broadcast_to(inv, r2.shape)], axis=1)


def _tc_post_body(sa_ref, sb_ref, r2_ref, b_ref, bt_ref,
                  wa1_ref, ba1_ref, wa2_ref, ba2_ref,
                  wc1_ref, bc1_ref, wc2_ref, bc2_ref,
                  mu_ref, val_ref):
    r2a = r2_ref[...]
    inv = r2a[:, 64:65]
    h2 = jnp.maximum((sa_ref[...] + sb_ref[...]) * inv + r2a[:, :64]
                     + b_ref[...], 0.0)
    n = h2.shape[0]
    bt = jnp.broadcast_to(bt_ref[...], (_G, n))
    ohT = (bt == lax.broadcasted_iota(jnp.int32, (_G, n), 0)) \
        .astype(jnp.float32)
    sums = _dot(ohT, h2)
    cnts = jnp.sum(ohT, axis=1, keepdims=True)
    pooled = sums / jnp.maximum(cnts, 1.0)
    a = jnp.maximum(_dot(pooled, wa1_ref[...]) + ba1_ref[...], 0.0)
    mu_ref[...] = _dot(a, wa2_ref[...]) + ba2_ref[...]
    cv = jnp.maximum(_dot(pooled, wc1_ref[...]) + bc1_ref[...], 0.0)
    val_ref[...] = _dot(cv, wc2_ref[...]) + bc2_ref[...]


def kernel(x, edge_index, batch, W1l, b1l, W1r, W2l, b2l, W2r,
           Wa1, ba1, Wa2, ba2, Wc1, bc1, Wc2, bc2):
    N, F = x.shape
    E = edge_index.shape[1]
    H = W1l.shape[0]
    A = Wa2.shape[0]
    W1 = H + 16
    C = 80

    f32 = jnp.float32
    nw = _N_CORES * _N_SUB
    src2d = edge_index[0].reshape(nw, E // (nw * C), C)
    dst2d = edge_index[1].reshape(nw, E // (nw * C), C)

    u1, r1 = pl.pallas_call(
        _tc_pre_body,
        out_shape=[jax.ShapeDtypeStruct((N, W1), f32),
                   jax.ShapeDtypeStruct((N, H), f32)],
    )(x, W1l.T, W1r.T)

    s1 = _make_sc_pass(N, E, W1, C)(u1, src2d, dst2d)

    u2, r2a = pl.pallas_call(
        _tc_mid_body,
        out_shape=[jax.ShapeDtypeStruct((N, H), f32),
                   jax.ShapeDtypeStruct((N, 2 * H), f32)],
    )(s1[0], s1[1], r1, b1l.reshape(1, H), W2l.T, W2r.T)

    s2 = _make_sc_pass(N, E, H, C)(u2, src2d, dst2d)

    mu, value = pl.pallas_call(
        _tc_post_body,
        out_shape=[jax.ShapeDtypeStruct((_G, A), f32),
                   jax.ShapeDtypeStruct((_G, 1), f32)],
    )(s2[0], s2[1], r2a, b2l.reshape(1, H), batch.reshape(1, N),
      Wa1.T, ba1.reshape(1, H), Wa2.T, ba2.reshape(1, A),
      Wc1.T, bc1.reshape(1, H), Wc2.T, bc2.reshape(1, 1))

    return (mu, value)
